# baseline (speedup 1.0000x reference)
import functools

import jax
import jax.numpy as jnp
from jax import lax
from jax.experimental import pallas as pl
from jax.experimental.pallas import tpu as pltpu
from jax.experimental.pallas import tpu_sc as plsc

BATCH = 4096
DIM = 64
SEQ = 50
NC, NS, L = 2, 16, 16
NW = NC * NS
BPW = BATCH // NW
RPG = 4
TPG = RPG * SEQ
GPW = BPW // RPG
NBUF = 4

_ROW_CHUNKS = (
    ((0, 0, 16), (16, 0, 16), (32, 0, 16), (48, 0, 2)),
    ((48, 2, 16), (64, 0, 16), (80, 0, 16), (96, 0, 4)),
    ((96, 4, 16), (112, 0, 16), (128, 0, 16), (144, 0, 6)),
    ((144, 6, 16), (160, 0, 16), (176, 0, 8), (184, 0, 16)),
)


def _sc_body(f0, f1, f2, tok, W0, W1, W2, Wt, out0, out1, out2, out3,
             idx0_v, idx1_v, idx2_v, r0_v, r1_v, r2_v,
             tok_v, w0_v, g_v, txt_v, sem_f, *sems):
    wid = lax.axis_index("s") * NC + lax.axis_index("c")
    base = wid * BPW

    pltpu.sync_copy(f0.at[pl.ds(base, BPW)], idx0_v)
    pltpu.sync_copy(f1.at[pl.ds(base, BPW)], idx1_v)
    pltpu.sync_copy(f2.at[pl.ds(base, BPW)], idx2_v)
    cp0 = pltpu.async_copy(W0.at[idx0_v], r0_v, sem_f)
    cp1 = pltpu.async_copy(W1.at[idx1_v], r1_v, sem_f)
    cp2 = pltpu.async_copy(W2.at[idx2_v], r2_v, sem_f)

    pltpu.sync_copy(tok.at[pl.ds(base * SEQ, BPW * SEQ)], tok_v)
    pltpu.sync_copy(Wt.at[pl.ds(0, 1)], w0_v)

    w0 = [w0_v[0, pl.ds(16 * k, 16)] for k in range(4)]
    lanes = lax.iota(jnp.int32, 16)

    def gather_group(g, buf, sem):
        goff = pl.multiple_of(g * TPG, 8)
        pltpu.async_copy(Wt.at[tok_v.at[pl.ds(goff, TPG)]],
                         g_v.at[buf], sem)

    def wait_group(buf, sem):
        pltpu.make_async_copy(Wt.at[pl.ds(0, TPG)], g_v.at[buf], sem).wait()

    def compute_group(g, buf):
        goff = pl.multiple_of(g * TPG, 8)
        for r in range(RPG):
            acc = [jnp.zeros((16,), jnp.float32) for _ in range(4)]
            for s in range(SEQ):
                row = r * SEQ + s
                for k in range(4):
                    acc[k] = acc[k] + g_v[buf, row, pl.ds(16 * k, 16)]
            nz = None
            for off, lo, hi in _ROW_CHUNKS[r]:
                ids = tok_v[pl.ds(goff + off, 16)]
                z = ids == 0
                if lo > 0:
                    z = z & (lanes >= lo)
                if hi < 16:
                    z = z & (lanes < hi)
                c = plsc.all_reduce_population_count(z)
                nz = c if nz is None else nz + c
            n0f = nz.astype(jnp.float32)
            denom = jnp.maximum(jnp.float32(SEQ) - n0f, 1.0)
            rcp = 1.0 / denom
            orow = g * RPG + r
            for k in range(4):
                e = (acc[k] - n0f * w0[k]) * rcp
                txt_v[orow, pl.ds(16 * k, 16)] = e

    for b in range(NBUF):
        gather_group(b, b, sems[b])

    def step(j, carry):
        for b in range(NBUF):
            g = j * NBUF + b
            wait_group(b, sems[b])
            compute_group(g, b)

            @pl.when(g + NBUF < GPW)
            def _():
                gather_group(g + NBUF, b, sems[b])

        return carry

    lax.fori_loop(0, GPW // NBUF, step, 0)

    cp0.wait()
    cp1.wait()
    cp2.wait()
    pltpu.sync_copy(r0_v, out0.at[pl.ds(base, BPW)])
    pltpu.sync_copy(r1_v, out1.at[pl.ds(base, BPW)])
    pltpu.sync_copy(r2_v, out2.at[pl.ds(base, BPW)])
    pltpu.sync_copy(txt_v, out3.at[pl.ds(base, BPW)])


@jax.jit
def _run(f0, f1, f2, tok, W0, W1, W2, Wt):
    mesh = plsc.VectorSubcoreMesh(core_axis_name="c", subcore_axis_name="s")
    return pl.kernel(
        _sc_body,
        out_type=[jax.ShapeDtypeStruct((BATCH, DIM), jnp.float32)] * 4,
        mesh=mesh,
        compiler_params=pltpu.CompilerParams(use_tc_tiling_on_sc=False,
                                             needs_layout_passes=False),
        scratch_types=[
            pltpu.VMEM((BPW,), jnp.int32),
            pltpu.VMEM((BPW,), jnp.int32),
            pltpu.VMEM((BPW,), jnp.int32),
            pltpu.VMEM((BPW, DIM), jnp.float32),
            pltpu.VMEM((BPW, DIM), jnp.float32),
            pltpu.VMEM((BPW, DIM), jnp.float32),
            pltpu.VMEM((BPW * SEQ,), jnp.int32),
            pltpu.VMEM((1, DIM), jnp.float32),
            pltpu.VMEM((NBUF, TPG, DIM), jnp.float32),
            pltpu.VMEM((BPW, DIM), jnp.float32),
            pltpu.SemaphoreType.DMA,
        ] + [pltpu.SemaphoreType.DMA] * NBUF,
    )(f0, f1, f2, tok, W0, W1, W2, Wt)


def kernel(f_str_0, f_str_1, f_int_0, f_text_0,
           W_str_0, W_str_1, W_int_0, W_text_0):
    tok = f_text_0.astype(jnp.int32).reshape(-1)
    e0, e1, e2, e3 = _run(f_str_0.astype(jnp.int32),
                          f_str_1.astype(jnp.int32),
                          f_int_0.astype(jnp.int32), tok,
                          W_str_0, W_str_1, W_int_0, W_text_0)
    return jnp.concatenate([e0, e1, e2, e3], axis=1)


# speedup vs baseline: 1.0278x; 1.0278x over previous
import functools

import jax
import jax.numpy as jnp
from jax import lax
from jax.experimental import pallas as pl
from jax.experimental.pallas import tpu as pltpu
from jax.experimental.pallas import tpu_sc as plsc

BATCH = 4096
DIM = 64
SEQ = 50
NC, NS, L = 2, 16, 16
NW = NC * NS
BPW = BATCH // NW
RPG = 4
TPG = RPG * SEQ
GPW = BPW // RPG
NBUF = 4

_ROW_CHUNKS = (
    ((0, 0, 16), (16, 0, 16), (32, 0, 16), (48, 0, 2)),
    ((48, 2, 16), (64, 0, 16), (80, 0, 16), (96, 0, 4)),
    ((96, 4, 16), (112, 0, 16), (128, 0, 16), (144, 0, 6)),
    ((144, 6, 16), (160, 0, 16), (176, 0, 8), (184, 0, 16)),
)


def _sc_body(f0, f1, f2, tok, W0, W1, W2, Wt, out,
             idx0_v, idx1_v, idx2_v, r0_v, r1_v, r2_v,
             tok_v, w0_v, g_v, txt_v, sem_f, *sems):
    wid = lax.axis_index("s") * NC + lax.axis_index("c")
    base = wid * BPW

    pltpu.sync_copy(f0.at[pl.ds(base, BPW)], idx0_v)
    pltpu.sync_copy(f1.at[pl.ds(base, BPW)], idx1_v)
    pltpu.sync_copy(f2.at[pl.ds(base, BPW)], idx2_v)
    cp0 = pltpu.async_copy(W0.at[idx0_v], r0_v, sem_f)
    cp1 = pltpu.async_copy(W1.at[idx1_v], r1_v, sem_f)
    cp2 = pltpu.async_copy(W2.at[idx2_v], r2_v, sem_f)

    pltpu.sync_copy(tok.at[pl.ds(base * SEQ, BPW * SEQ)], tok_v)
    pltpu.sync_copy(Wt.at[pl.ds(0, 1)], w0_v)

    w0 = [w0_v[0, pl.ds(16 * k, 16)] for k in range(4)]
    lanes = lax.iota(jnp.int32, 16)

    def gather_group(g, buf, sem):
        goff = pl.multiple_of(g * TPG, 8)
        pltpu.async_copy(Wt.at[tok_v.at[pl.ds(goff, TPG)]],
                         g_v.at[buf], sem)

    def wait_group(buf, sem):
        pltpu.make_async_copy(Wt.at[pl.ds(0, TPG)], g_v.at[buf], sem).wait()

    def compute_group(g, buf):
        goff = pl.multiple_of(g * TPG, 8)
        for r in range(RPG):
            acc = [jnp.zeros((16,), jnp.float32) for _ in range(4)]
            for s in range(SEQ):
                row = r * SEQ + s
                for k in range(4):
                    acc[k] = acc[k] + g_v[buf, row, pl.ds(16 * k, 16)]
            nz = None
            for off, lo, hi in _ROW_CHUNKS[r]:
                ids = tok_v[pl.ds(goff + off, 16)]
                z = ids == 0
                if lo > 0:
                    z = z & (lanes >= lo)
                if hi < 16:
                    z = z & (lanes < hi)
                c = plsc.all_reduce_population_count(z)
                nz = c if nz is None else nz + c
            n0f = nz.astype(jnp.float32)
            denom = jnp.maximum(jnp.float32(SEQ) - n0f, 1.0)
            rcp = 1.0 / denom
            orow = g * RPG + r
            for k in range(4):
                e = (acc[k] - n0f * w0[k]) * rcp
                txt_v[orow, pl.ds(16 * k, 16)] = e

    for b in range(NBUF):
        gather_group(b, b, sems[b])

    def step(j, carry):
        for b in range(NBUF):
            g = j * NBUF + b
            wait_group(b, sems[b])
            compute_group(g, b)

            @pl.when(g + NBUF < GPW)
            def _():
                gather_group(g + NBUF, b, sems[b])

        return carry

    lax.fori_loop(0, GPW // NBUF, step, 0)

    cp0.wait()
    cp1.wait()
    cp2.wait()
    pltpu.sync_copy(r0_v, out.at[pl.ds(base, BPW), pl.ds(0, DIM)])
    pltpu.sync_copy(r1_v, out.at[pl.ds(base, BPW), pl.ds(DIM, DIM)])
    pltpu.sync_copy(r2_v, out.at[pl.ds(base, BPW), pl.ds(2 * DIM, DIM)])
    pltpu.sync_copy(txt_v, out.at[pl.ds(base, BPW), pl.ds(3 * DIM, DIM)])


@jax.jit
def _run(f0, f1, f2, tok, W0, W1, W2, Wt):
    mesh = plsc.VectorSubcoreMesh(core_axis_name="c", subcore_axis_name="s")
    return pl.kernel(
        _sc_body,
        out_type=jax.ShapeDtypeStruct((BATCH, 4 * DIM), jnp.float32),
        mesh=mesh,
        compiler_params=pltpu.CompilerParams(use_tc_tiling_on_sc=False,
                                             needs_layout_passes=False),
        scratch_types=[
            pltpu.VMEM((BPW,), jnp.int32),
            pltpu.VMEM((BPW,), jnp.int32),
            pltpu.VMEM((BPW,), jnp.int32),
            pltpu.VMEM((BPW, DIM), jnp.float32),
            pltpu.VMEM((BPW, DIM), jnp.float32),
            pltpu.VMEM((BPW, DIM), jnp.float32),
            pltpu.VMEM((BPW * SEQ,), jnp.int32),
            pltpu.VMEM((1, DIM), jnp.float32),
            pltpu.VMEM((NBUF, TPG, DIM), jnp.float32),
            pltpu.VMEM((BPW, DIM), jnp.float32),
            pltpu.SemaphoreType.DMA,
        ] + [pltpu.SemaphoreType.DMA] * NBUF,
    )(f0, f1, f2, tok, W0, W1, W2, Wt)


def kernel(f_str_0, f_str_1, f_int_0, f_text_0,
           W_str_0, W_str_1, W_int_0, W_text_0):
    tok = f_text_0.astype(jnp.int32).reshape(-1)
    return _run(f_str_0.astype(jnp.int32), f_str_1.astype(jnp.int32),
                f_int_0.astype(jnp.int32), tok,
                W_str_0, W_str_1, W_int_0, W_text_0)


# speedup vs baseline: 1.2993x; 1.2642x over previous
import functools

import jax
import jax.numpy as jnp
from jax import lax
from jax.experimental import pallas as pl
from jax.experimental.pallas import tpu as pltpu
from jax.experimental.pallas import tpu_sc as plsc

BATCH = 4096
DIM = 64
SEQ = 50
NC, NS, L = 2, 16, 16
NW = NC * NS
BPW = BATCH // NW
RPG = 4
TPG = RPG * SEQ
GPW = BPW // RPG
NBUF = 4

_ROW_CHUNKS = (
    ((0, 0, 16), (16, 0, 16), (32, 0, 16), (48, 0, 2)),
    ((48, 2, 16), (64, 0, 16), (80, 0, 16), (96, 0, 4)),
    ((96, 4, 16), (112, 0, 16), (128, 0, 16), (144, 0, 6)),
    ((144, 6, 16), (160, 0, 16), (176, 0, 8), (184, 0, 16)),
)


def _text_body(tok, Wt, out, tok_v, w0_v, g_v, txt_v, *sems):
    wid = lax.axis_index("s") * NC + lax.axis_index("c")
    base = wid * BPW

    pltpu.sync_copy(tok.at[pl.ds(base * SEQ, BPW * SEQ)], tok_v)
    pltpu.sync_copy(Wt.at[pl.ds(0, 1)], w0_v)

    w0 = [w0_v[0, pl.ds(16 * k, 16)] for k in range(4)]
    lanes = lax.iota(jnp.int32, 16)

    def gather_group(g, buf, sem):
        goff = pl.multiple_of(g * TPG, 8)
        pltpu.async_copy(Wt.at[tok_v.at[pl.ds(goff, TPG)]],
                         g_v.at[buf], sem)

    def wait_group(buf, sem):
        pltpu.make_async_copy(Wt.at[pl.ds(0, TPG)], g_v.at[buf], sem).wait()

    def compute_group(g, buf):
        goff = pl.multiple_of(g * TPG, 8)
        for r in range(RPG):
            acc = [jnp.zeros((16,), jnp.float32) for _ in range(4)]
            for s in range(SEQ):
                row = r * SEQ + s
                for k in range(4):
                    acc[k] = acc[k] + g_v[buf, row, pl.ds(16 * k, 16)]
            nz = None
            for off, lo, hi in _ROW_CHUNKS[r]:
                ids = tok_v[pl.ds(goff + off, 16)]
                z = ids == 0
                if lo > 0:
                    z = z & (lanes >= lo)
                if hi < 16:
                    z = z & (lanes < hi)
                c = plsc.all_reduce_population_count(z)
                nz = c if nz is None else nz + c
            n0f = nz.astype(jnp.float32)
            denom = jnp.maximum(jnp.float32(SEQ) - n0f, 1.0)
            rcp = 1.0 / denom
            orow = g * RPG + r
            for k in range(4):
                e = (acc[k] - n0f * w0[k]) * rcp
                txt_v[orow, pl.ds(16 * k, 16)] = e

    for b in range(NBUF):
        gather_group(b, b, sems[b])

    def step(j, carry):
        for b in range(NBUF):
            g = j * NBUF + b
            wait_group(b, sems[b])
            compute_group(g, b)

            @pl.when(g + NBUF < GPW)
            def _():
                gather_group(g + NBUF, b, sems[b])

        return carry

    lax.fori_loop(0, GPW // NBUF, step, 0)

    pltpu.sync_copy(txt_v, out.at[pl.ds(base, BPW)])


def _plain_body(f0, f1, f2, W0, W1, W2, out0, out1, out2,
                idx0_v, idx1_v, idx2_v, r0_v, r1_v, r2_v, sem_f):
    wid = lax.axis_index("s") * NC + lax.axis_index("c")
    base = wid * BPW

    pltpu.sync_copy(f0.at[pl.ds(base, BPW)], idx0_v)
    pltpu.sync_copy(f1.at[pl.ds(base, BPW)], idx1_v)
    pltpu.sync_copy(f2.at[pl.ds(base, BPW)], idx2_v)
    cp0 = pltpu.async_copy(W0.at[idx0_v], r0_v, sem_f)
    cp1 = pltpu.async_copy(W1.at[idx1_v], r1_v, sem_f)
    cp2 = pltpu.async_copy(W2.at[idx2_v], r2_v, sem_f)
    cp0.wait()
    cp1.wait()
    cp2.wait()
    pltpu.sync_copy(r0_v, out0.at[pl.ds(base, BPW)])
    pltpu.sync_copy(r1_v, out1.at[pl.ds(base, BPW)])
    pltpu.sync_copy(r2_v, out2.at[pl.ds(base, BPW)])


@jax.jit
def _run(f0, f1, f2, tok, W0, W1, W2, Wt):
    mesh = plsc.VectorSubcoreMesh(core_axis_name="c", subcore_axis_name="s")
    params = pltpu.CompilerParams(use_tc_tiling_on_sc=False,
                                  needs_layout_passes=False)
    e_text = pl.kernel(
        _text_body,
        out_type=jax.ShapeDtypeStruct((BATCH, DIM), jnp.float32),
        mesh=mesh,
        compiler_params=params,
        scratch_types=[
            pltpu.VMEM((BPW * SEQ,), jnp.int32),
            pltpu.VMEM((1, DIM), jnp.float32),
            pltpu.VMEM((NBUF, TPG, DIM), jnp.float32),
            pltpu.VMEM((BPW, DIM), jnp.float32),
        ] + [pltpu.SemaphoreType.DMA] * NBUF,
    )(tok, Wt)
    e0, e1, e2 = pl.kernel(
        _plain_body,
        out_type=[jax.ShapeDtypeStruct((BATCH, DIM), jnp.float32)] * 3,
        mesh=mesh,
        compiler_params=params,
        scratch_types=[
            pltpu.VMEM((BPW,), jnp.int32),
            pltpu.VMEM((BPW,), jnp.int32),
            pltpu.VMEM((BPW,), jnp.int32),
            pltpu.VMEM((BPW, DIM), jnp.float32),
            pltpu.VMEM((BPW, DIM), jnp.float32),
            pltpu.VMEM((BPW, DIM), jnp.float32),
            pltpu.SemaphoreType.DMA,
        ],
    )(f0, f1, f2, W0, W1, W2)
    return jnp.concatenate([e0, e1, e2, e_text], axis=1)


def kernel(f_str_0, f_str_1, f_int_0, f_text_0,
           W_str_0, W_str_1, W_int_0, W_text_0):
    tok = f_text_0.astype(jnp.int32).reshape(-1)
    return _run(f_str_0.astype(jnp.int32), f_str_1.astype(jnp.int32),
                f_int_0.astype(jnp.int32), tok,
                W_str_0, W_str_1, W_int_0, W_text_0)
